# Initial kernel scaffold; baseline (speedup 1.0000x reference)
import jax
import jax.numpy as jnp
from jax.experimental import pallas as pl


def kernel(x, edge_index, W_self0, W_neigh0, b0, W_self1, W_neigh1, b1):
    raise NotImplementedError("write your pallas kernel here")


# speedup vs baseline: 5.9856x; 5.9856x over previous
import functools

import jax
import jax.numpy as jnp
from jax import lax
from jax.experimental import pallas as pl
from jax.experimental.pallas import tpu as pltpu
from jax.experimental.pallas import tpu_sc as plsc

_NC = 2
_NS = 16


def _segment_sum_sc(table, src, dst, zeros):
    n_rows, width = table.shape
    n_edges = src.shape[0]
    epw = n_edges // (_NC * _NS)
    chunk = 80
    n_chunks = epw // chunk
    assert epw % chunk == 0
    rpt = n_rows // _NS
    assert n_rows % _NS == 0

    mesh = plsc.VectorSubcoreMesh(core_axis_name="c", subcore_axis_name="s")

    @functools.partial(
        pl.kernel,
        out_type=jax.ShapeDtypeStruct((_NC, n_rows, width), jnp.float32),
        mesh=mesh,
        scratch_types=[
            pltpu.VMEM((chunk,), jnp.int32),
            pltpu.VMEM((chunk,), jnp.int32),
            pltpu.VMEM((chunk, width), jnp.float32),
            pltpu.VMEM_SHARED((n_rows, width), jnp.float32),
            pltpu.SemaphoreType.DMA,
        ],
        compiler_params=pltpu.CompilerParams(use_tc_tiling_on_sc=False),
    )
    def seg_sum(table_hbm, src_hbm, dst_hbm, zeros_hbm, out_hbm,
                sidx, didx, rows, accum, sem):
        c = lax.axis_index("c")
        s = lax.axis_index("s")
        pltpu.sync_copy(zeros_hbm.at[pl.ds(s * rpt, rpt)],
                        accum.at[pl.ds(s * rpt, rpt)])
        plsc.subcore_barrier()

        base_e = (c * _NS + s) * epw

        def body(i, carry):
            b = base_e + i * chunk
            pltpu.sync_copy(src_hbm.at[pl.ds(b, chunk)], sidx)
            pltpu.sync_copy(dst_hbm.at[pl.ds(b, chunk)], didx)
            pltpu.async_copy(table_hbm.at[sidx], rows, sem).wait()
            pltpu.sync_copy(rows, accum.at[didx], add=True)
            return carry

        lax.fori_loop(0, n_chunks, body, 0, unroll=False)
        plsc.subcore_barrier()
        pltpu.sync_copy(accum.at[pl.ds(s * rpt, rpt)],
                        out_hbm.at[c, pl.ds(s * rpt, rpt)])

    return seg_sum(table, src, dst, zeros)


def _mid_tc(x, p0, w_self0, w_neigh0, b0, w_neigh1):
    n, d_in = x.shape
    d_hid = w_self0.shape[1]
    d_out = w_neigh1.shape[1]
    wdeg = p0.shape[2]
    blk = 1000
    grid = n // blk

    def body(x_ref, p_ref, ws_ref, wn_ref, b_ref, wn1_ref,
             h1_ref, g_ref, invd_ref):
        acc = p_ref[0] + p_ref[1]
        inv = 1.0 / jnp.maximum(acc[:, d_in:d_in + 1], 1.0)
        hn = acc[:, :d_in] * inv
        h1 = x_ref[...] @ ws_ref[...] + hn @ wn_ref[...] + b_ref[...]
        h1 = jnp.maximum(h1, 0.0)
        h1_ref[...] = h1
        g_ref[...] = h1 @ wn1_ref[...]
        invd_ref[...] = inv

    return pl.pallas_call(
        body,
        grid=(grid,),
        in_specs=[
            pl.BlockSpec((blk, d_in), lambda i: (i, 0)),
            pl.BlockSpec((_NC, blk, wdeg), lambda i: (0, i, 0)),
            pl.BlockSpec((d_in, d_hid), lambda i: (0, 0)),
            pl.BlockSpec((d_in, d_hid), lambda i: (0, 0)),
            pl.BlockSpec((1, d_hid), lambda i: (0, 0)),
            pl.BlockSpec((d_hid, d_out), lambda i: (0, 0)),
        ],
        out_specs=[
            pl.BlockSpec((blk, d_hid), lambda i: (i, 0)),
            pl.BlockSpec((blk, d_out), lambda i: (i, 0)),
            pl.BlockSpec((blk, 1), lambda i: (i, 0)),
        ],
        out_shape=[
            jax.ShapeDtypeStruct((n, d_hid), jnp.float32),
            jax.ShapeDtypeStruct((n, d_out), jnp.float32),
            jax.ShapeDtypeStruct((n, 1), jnp.float32),
        ],
    )(x, p0, w_self0, w_neigh0, b0, w_neigh1)


def _final_tc(h1, p1, inv_deg, w_self1, b1):
    n, d_hid = h1.shape
    d_out = w_self1.shape[1]
    blk = 1000
    grid = n // blk

    def body(h_ref, p_ref, invd_ref, ws_ref, b_ref, out_ref):
        agg = (p_ref[0] + p_ref[1]) * invd_ref[...]
        out_ref[...] = h_ref[...] @ ws_ref[...] + agg + b_ref[...]

    return pl.pallas_call(
        body,
        grid=(grid,),
        in_specs=[
            pl.BlockSpec((blk, d_hid), lambda i: (i, 0)),
            pl.BlockSpec((_NC, blk, d_out), lambda i: (0, i, 0)),
            pl.BlockSpec((blk, 1), lambda i: (i, 0)),
            pl.BlockSpec((d_hid, d_out), lambda i: (0, 0)),
            pl.BlockSpec((1, d_out), lambda i: (0, 0)),
        ],
        out_specs=pl.BlockSpec((blk, d_out), lambda i: (i, 0)),
        out_shape=jax.ShapeDtypeStruct((n, d_out), jnp.float32),
    )(h1, p1, inv_deg, w_self1, b1)


def kernel(x, edge_index, W_self0, W_neigh0, b0, W_self1, W_neigh1, b1):
    n, d_in = x.shape
    src = edge_index[0]
    dst = edge_index[1]

    wdeg = d_in + 16
    xpad = jnp.concatenate(
        [x, jnp.ones((n, 1), jnp.float32), jnp.zeros((n, 15), jnp.float32)],
        axis=1)
    zeros_wide = jnp.zeros((n, wdeg), jnp.float32)

    p0 = _segment_sum_sc(xpad, src, dst, zeros_wide)
    h1, g, inv_deg = _mid_tc(x, p0, W_self0, W_neigh0,
                             b0.reshape(1, -1), W_neigh1)
    d_out = g.shape[1]
    p1 = _segment_sum_sc(g, src, dst, zeros_wide[:, :d_out])
    return _final_tc(h1, p1, inv_deg, W_self1, b1.reshape(1, -1))


# speedup vs baseline: 13.6825x; 2.2859x over previous
import functools

import jax
import jax.numpy as jnp
from jax import lax
from jax.experimental import pallas as pl
from jax.experimental.pallas import tpu as pltpu
from jax.experimental.pallas import tpu_sc as plsc

_NC = 2
_NS = 16


_DW = 16


def _segment_sum_sc(table, src, dst, zeros, deg_aux=None):
    n_rows, width = table.shape
    n_chunks, chunk = src.shape
    cpw = n_chunks // (_NC * _NS)
    rpt = n_rows // _NS
    assert n_rows % _NS == 0 and n_chunks % (_NC * _NS) == 0
    n_pairs = (cpw - 1) // 2
    assert cpw == 2 * n_pairs + 1
    with_deg = deg_aux is not None

    mesh = plsc.VectorSubcoreMesh(core_axis_name="c", subcore_axis_name="s")

    out_type = [jax.ShapeDtypeStruct((_NC, n_rows, width), jnp.float32)]
    scratch = [
        pltpu.VMEM((chunk,), jnp.int32),
        pltpu.VMEM((chunk,), jnp.int32),
        pltpu.VMEM((cpw, chunk), jnp.int32),
        pltpu.VMEM((chunk, width), jnp.float32),
        pltpu.VMEM((chunk, width), jnp.float32),
        pltpu.VMEM_SHARED((n_rows, width), jnp.float32),
        pltpu.SemaphoreType.DMA,
        pltpu.SemaphoreType.DMA,
        pltpu.SemaphoreType.DMA,
        pltpu.SemaphoreType.DMA,
    ]
    if with_deg:
        out_type.append(jax.ShapeDtypeStruct((_NC, n_rows, _DW), jnp.float32))
        scratch += [
            pltpu.VMEM((chunk, _DW), jnp.float32),
            pltpu.VMEM_SHARED((n_rows, _DW), jnp.float32),
        ]

    @functools.partial(
        pl.kernel,
        out_type=out_type,
        mesh=mesh,
        scratch_types=scratch,
        compiler_params=pltpu.CompilerParams(use_tc_tiling_on_sc=False),
    )
    def seg_sum(*refs):
        if with_deg:
            (table_hbm, src_hbm, dst_hbm, zeros_hbm, zd_hbm, ones_hbm,
             out_hbm, outd_hbm,
             sidx0, sidx1, didx, rows0, rows1, accum,
             gsem0, gsem1, isem0, isem1,
             ones_rows, dacc) = refs
        else:
            (table_hbm, src_hbm, dst_hbm, zeros_hbm, out_hbm,
             sidx0, sidx1, didx, rows0, rows1, accum,
             gsem0, gsem1, isem0, isem1) = refs
        c = lax.axis_index("c")
        s = lax.axis_index("s")
        wid = c * _NS + s
        base = wid * cpw
        pltpu.sync_copy(zeros_hbm.at[pl.ds(s * rpt, rpt)],
                        accum.at[pl.ds(s * rpt, rpt)])
        if with_deg:
            pltpu.sync_copy(zd_hbm.at[pl.ds(s * rpt, rpt)],
                            dacc.at[pl.ds(s * rpt, rpt)])
            pltpu.sync_copy(ones_hbm, ones_rows)
        pltpu.sync_copy(dst_hbm.at[pl.ds(base, cpw)], didx)
        pltpu.sync_copy(src_hbm.at[base], sidx0)
        pltpu.sync_copy(src_hbm.at[base + 1], sidx1)
        plsc.subcore_barrier()

        def gather(sidx, rows, sem):
            return pltpu.async_copy(table_hbm.at[sidx], rows, sem)

        def scatter(n, rows):
            pltpu.sync_copy(rows, accum.at[didx.at[n]], add=True)
            if with_deg:
                pltpu.sync_copy(ones_rows, dacc.at[didx.at[n]], add=True)

        gather(sidx0, rows0, gsem0)
        gather(sidx1, rows1, gsem1)

        def half(n, sidx, rows, gsem, isem):
            pltpu.make_async_copy(table_hbm.at[sidx], rows, gsem).wait()

            @pl.when(n + 2 < cpw)
            def _():
                pltpu.async_copy(src_hbm.at[base + n + 2], sidx, isem)

            scatter(n, rows)

            @pl.when(n + 2 < cpw)
            def _():
                pltpu.make_async_copy(src_hbm.at[base + n + 2], sidx,
                                      isem).wait()
                gather(sidx, rows, gsem)

        def body(m, carry):
            n0 = 2 * m
            half(n0, sidx0, rows0, gsem0, isem0)
            half(n0 + 1, sidx1, rows1, gsem1, isem1)
            return carry

        lax.fori_loop(0, n_pairs, body, 0, unroll=False)
        half(cpw - 1, sidx0, rows0, gsem0, isem0)
        plsc.subcore_barrier()
        pltpu.sync_copy(accum.at[pl.ds(s * rpt, rpt)],
                        out_hbm.at[c, pl.ds(s * rpt, rpt)])
        if with_deg:
            pltpu.sync_copy(dacc.at[pl.ds(s * rpt, rpt)],
                            outd_hbm.at[c, pl.ds(s * rpt, rpt)])

    if with_deg:
        return seg_sum(table, src, dst, zeros, *deg_aux)
    return seg_sum(table, src, dst, zeros)


def _mid_tc(x, p0, pd, w_self0, w_neigh0, b0, w_neigh1):
    n, d_in = x.shape
    d_hid = w_self0.shape[1]
    d_out = w_neigh1.shape[1]
    blk = 1000
    grid = n // blk

    def body(x_ref, p_ref, pd_ref, ws_ref, wn_ref, b_ref, wn1_ref,
             h1_ref, g_ref, invd_ref):
        acc = p_ref[0] + p_ref[1]
        deg = pd_ref[0, :, 0:1] + pd_ref[1, :, 0:1]
        inv = 1.0 / jnp.maximum(deg, 1.0)
        hn = acc * inv
        h1 = x_ref[...] @ ws_ref[...] + hn @ wn_ref[...] + b_ref[...]
        h1 = jnp.maximum(h1, 0.0)
        h1_ref[...] = h1
        g_ref[...] = h1 @ wn1_ref[...]
        invd_ref[...] = inv

    return pl.pallas_call(
        body,
        grid=(grid,),
        in_specs=[
            pl.BlockSpec((blk, d_in), lambda i: (i, 0)),
            pl.BlockSpec((_NC, blk, d_in), lambda i: (0, i, 0)),
            pl.BlockSpec((_NC, blk, _DW), lambda i: (0, i, 0)),
            pl.BlockSpec((d_in, d_hid), lambda i: (0, 0)),
            pl.BlockSpec((d_in, d_hid), lambda i: (0, 0)),
            pl.BlockSpec((1, d_hid), lambda i: (0, 0)),
            pl.BlockSpec((d_hid, d_out), lambda i: (0, 0)),
        ],
        out_specs=[
            pl.BlockSpec((blk, d_hid), lambda i: (i, 0)),
            pl.BlockSpec((blk, d_out), lambda i: (i, 0)),
            pl.BlockSpec((blk, 1), lambda i: (i, 0)),
        ],
        out_shape=[
            jax.ShapeDtypeStruct((n, d_hid), jnp.float32),
            jax.ShapeDtypeStruct((n, d_out), jnp.float32),
            jax.ShapeDtypeStruct((n, 1), jnp.float32),
        ],
    )(x, p0, pd, w_self0, w_neigh0, b0, w_neigh1)


def _final_tc(h1, p1, inv_deg, w_self1, b1):
    n, d_hid = h1.shape
    d_out = w_self1.shape[1]
    blk = 1000
    grid = n // blk

    def body(h_ref, p_ref, invd_ref, ws_ref, b_ref, out_ref):
        agg = (p_ref[0] + p_ref[1]) * invd_ref[...]
        out_ref[...] = h_ref[...] @ ws_ref[...] + agg + b_ref[...]

    return pl.pallas_call(
        body,
        grid=(grid,),
        in_specs=[
            pl.BlockSpec((blk, d_hid), lambda i: (i, 0)),
            pl.BlockSpec((_NC, blk, d_out), lambda i: (0, i, 0)),
            pl.BlockSpec((blk, 1), lambda i: (i, 0)),
            pl.BlockSpec((d_hid, d_out), lambda i: (0, 0)),
            pl.BlockSpec((1, d_out), lambda i: (0, 0)),
        ],
        out_specs=pl.BlockSpec((blk, d_out), lambda i: (i, 0)),
        out_shape=jax.ShapeDtypeStruct((n, d_out), jnp.float32),
    )(h1, p1, inv_deg, w_self1, b1)


def kernel(x, edge_index, W_self0, W_neigh0, b0, W_self1, W_neigh1, b1):
    n, d_in = x.shape
    chunk = 80
    src = edge_index[0].reshape(-1, chunk)
    dst = edge_index[1].reshape(-1, chunk)

    zeros = jnp.zeros((n, d_in), jnp.float32)
    zeros_d = jnp.zeros((n, _DW), jnp.float32)
    ones_blk = jnp.ones((chunk, _DW), jnp.float32)

    p0, pd = _segment_sum_sc(x, src, dst, zeros,
                             deg_aux=(zeros_d, ones_blk))
    h1, g, inv_deg = _mid_tc(x, p0, pd, W_self0, W_neigh0,
                             b0.reshape(1, -1), W_neigh1)
    p1, = _segment_sum_sc(g, src, dst, zeros)
    return _final_tc(h1, p1, inv_deg, W_self1, b1.reshape(1, -1))
